# baseline (speedup 1.0000x reference)
import functools

import jax
import jax.numpy as jnp
from jax import lax
from jax.experimental import pallas as pl
from jax.experimental.pallas import tpu as pltpu
from jax.experimental.pallas import tpu_sc as plsc

L = 2048
N = L * L
NC = 2
NS = 16
NW = NC * NS
PER_W = N // NW
CHUNK = 4096
NCHUNK = PER_W // CHUNK
GROUPS = CHUNK // 16


def _sc_body(q_hbm, a_hbm, b_hbm, p_hbm, al_hbm, ga_hbm, out_hbm,
             qv, av, bv, pv, alv, gav):
    wid = lax.axis_index("s") * NC + lax.axis_index("c")
    base = wid * PER_W

    pltpu.sync_copy(al_hbm, alv)
    pltpu.sync_copy(ga_hbm, gav)
    alpha = alv[...]
    gamma = gav[...]

    lane = jnp.arange(16, dtype=jnp.int32)

    def do_chunk(c, _):
        off = base + c * CHUNK
        pltpu.sync_copy(q_hbm.at[pl.ds(off * 4, CHUNK * 4)], qv)
        pltpu.sync_copy(a_hbm.at[pl.ds(off, CHUNK)], av)
        pltpu.sync_copy(b_hbm.at[pl.ds(off, CHUNK)], bv)
        pltpu.sync_copy(p_hbm.at[pl.ds(off, CHUNK)], pv)

        def do_group(g, _):
            s = g * 16
            ii4 = (lane + s) * 4
            a = av[pl.ds(s, 16)]
            b = bv[pl.ds(s, 16)]
            p = pv[pl.ds(s, 16)]
            ib = ii4 + b * 2
            q_b0 = plsc.load_gather(qv, [ib])
            q_b1 = plsc.load_gather(qv, [ib + 1])
            ic = ii4 + a * 2 + b
            cur = plsc.load_gather(qv, [ic])
            maxv = jnp.maximum(q_b0, q_b1)
            upd = cur + alpha * (p + gamma * maxv - cur)
            plsc.store_scatter(qv, [ic], upd)
            return 0

        lax.fori_loop(0, GROUPS, do_group, 0, unroll=2)
        pltpu.sync_copy(qv, out_hbm.at[pl.ds(off * 4, CHUNK * 4)])
        return 0

    lax.fori_loop(0, NCHUNK, do_chunk, 0)


@jax.jit
def _run(q2, a_flat, b_flat, p_flat, al16, ga16):
    mesh = plsc.VectorSubcoreMesh(core_axis_name="c", subcore_axis_name="s")
    fn = functools.partial(
        pl.kernel,
        mesh=mesh,
        compiler_params=pltpu.CompilerParams(needs_layout_passes=False),
        out_type=jax.ShapeDtypeStruct((N * 4,), jnp.float32),
        scratch_types=[
            pltpu.VMEM((CHUNK * 4,), jnp.float32),
            pltpu.VMEM((CHUNK,), jnp.int32),
            pltpu.VMEM((CHUNK,), jnp.int32),
            pltpu.VMEM((CHUNK,), jnp.float32),
            pltpu.VMEM((16,), jnp.float32),
            pltpu.VMEM((16,), jnp.float32),
        ],
    )(_sc_body)
    return fn(q2, a_flat, b_flat, p_flat, al16, ga16)


def kernel(alpha, gamma, type_t_matrix, type_t1_matrix, Q_tensor, profit_matrix):
    a_flat = type_t_matrix.reshape(N).astype(jnp.int32)
    b_flat = type_t1_matrix.reshape(N).astype(jnp.int32)
    p_flat = profit_matrix.reshape(N).astype(jnp.float32)
    q2 = Q_tensor.reshape(N * 4)
    al16 = jnp.full((16,), alpha, jnp.float32)
    ga16 = jnp.full((16,), gamma, jnp.float32)
    out = _run(q2, a_flat, b_flat, p_flat, al16, ga16)
    return out.reshape(N, 2, 2)


# speedup vs baseline: 112.5793x; 112.5793x over previous
import functools

import jax
import jax.numpy as jnp
from jax import lax
from jax.experimental import pallas as pl
from jax.experimental.pallas import tpu as pltpu
from jax.experimental.pallas import tpu_sc as plsc

L = 2048
N = L * L
NC = 2
NS = 16
NW = NC * NS
NCHUNK = 1024
PER_W = NCHUNK // NW


def _sc_body(q_hbm, a_hbm, b_hbm, p_hbm, al_hbm, ga_hbm, out_hbm,
             av, bv, pv, qv, ov, alv, gav):
    wid = lax.axis_index("s") * NC + lax.axis_index("c")

    pltpu.sync_copy(al_hbm, alv)
    pltpu.sync_copy(ga_hbm, gav)
    alpha = alv[...]
    gamma = gav[...]

    def do_chunk(k, _):
        cid = wid * PER_W + k
        r = cid // 4
        cc = (cid % 4) * 4
        pltpu.sync_copy(a_hbm.at[r, pl.ds(cc, 4), :, :], av)
        pltpu.sync_copy(b_hbm.at[r, pl.ds(cc, 4), :, :], bv)
        pltpu.sync_copy(p_hbm.at[r, pl.ds(cc, 4), :, :], pv)
        pltpu.sync_copy(q_hbm.at[:, r, :, pl.ds(cc, 4), :, :], qv)

        def do_pc(t, _):
            p = t // 4
            c = t % 4
            for j in range(8):
                s = j * 16
                a16 = av[c, p, pl.ds(s, 16)]
                b16 = bv[c, p, pl.ds(s, 16)]
                pr = pv[c, p, pl.ds(s, 16)]
                q00 = qv[0, p, c, 0, pl.ds(s, 16)]
                q01 = qv[0, p, c, 1, pl.ds(s, 16)]
                q10 = qv[1, p, c, 0, pl.ds(s, 16)]
                q11 = qv[1, p, c, 1, pl.ds(s, 16)]
                ca = a16 == 0
                cb = b16 == 0
                maxv = jnp.where(cb, jnp.maximum(q00, q01),
                                 jnp.maximum(q10, q11))
                cur = jnp.where(ca, jnp.where(cb, q00, q01),
                                jnp.where(cb, q10, q11))
                upd = cur + alpha * (pr + gamma * maxv - cur)
                ov[0, p, c, 0, pl.ds(s, 16)] = jnp.where(ca & cb, upd, q00)
                ov[0, p, c, 1, pl.ds(s, 16)] = jnp.where(ca & ~cb, upd, q01)
                ov[1, p, c, 0, pl.ds(s, 16)] = jnp.where(~ca & cb, upd, q10)
                ov[1, p, c, 1, pl.ds(s, 16)] = jnp.where(~ca & ~cb, upd, q11)
            return 0

        lax.fori_loop(0, 32, do_pc, 0)
        pltpu.sync_copy(ov, out_hbm.at[:, r, :, pl.ds(cc, 4), :, :])
        return 0

    lax.fori_loop(0, PER_W, do_chunk, 0)


@jax.jit
def _run(q6, sa, sb, sp, al16, ga16):
    mesh = plsc.VectorSubcoreMesh(core_axis_name="c", subcore_axis_name="s")
    fn = functools.partial(
        pl.kernel,
        mesh=mesh,
        compiler_params=pltpu.CompilerParams(needs_layout_passes=False),
        out_type=jax.ShapeDtypeStruct((2, 256, 8, 16, 2, 128), jnp.float32),
        scratch_types=[
            pltpu.VMEM((4, 8, 128), jnp.int32),
            pltpu.VMEM((4, 8, 128), jnp.int32),
            pltpu.VMEM((4, 8, 128), jnp.float32),
            pltpu.VMEM((2, 8, 4, 2, 128), jnp.float32),
            pltpu.VMEM((2, 8, 4, 2, 128), jnp.float32),
            pltpu.VMEM((16,), jnp.float32),
            pltpu.VMEM((16,), jnp.float32),
        ],
    )(_sc_body)
    return fn(q6, sa, sb, sp, al16, ga16)


def _tile_view(m):
    return m.reshape(256, 8, 16, 128).transpose(0, 2, 1, 3)


def kernel(alpha, gamma, type_t_matrix, type_t1_matrix, Q_tensor, profit_matrix):
    sa = _tile_view(type_t_matrix.astype(jnp.int32))
    sb = _tile_view(type_t1_matrix.astype(jnp.int32))
    sp = _tile_view(profit_matrix.astype(jnp.float32))
    q6 = (Q_tensor.reshape(32768, 128, 2, 2)
          .transpose(2, 0, 3, 1)
          .reshape(2, 256, 8, 16, 2, 128))
    al16 = jnp.full((16,), alpha, jnp.float32)
    ga16 = jnp.full((16,), gamma, jnp.float32)
    out = _run(q6, sa, sb, sp, al16, ga16)
    return (out.reshape(2, 32768, 2, 128)
            .transpose(1, 3, 0, 2)
            .reshape(N, 2, 2))
